# baseline (speedup 1.0000x reference)
import jax
import jax.numpy as jnp
from jax.experimental import pallas as pl
from jax.experimental.pallas import tpu as pltpu


def _lrelu(x):
    return jnp.maximum(x, 0.01 * x)


def _lv_body(x_ref, lvW1_ref, lvb1_ref, lvW2_ref, lvb2_ref, lv_ref, x16_ref):
    f32 = jnp.float32
    bf16 = jnp.bfloat16
    x = x_ref[...].astype(bf16)
    x16_ref[...] = x
    g = jnp.dot(x, lvW1_ref[...].astype(bf16), preferred_element_type=f32) + lvb1_ref[...]
    g = _lrelu(g).astype(bf16)
    lv_ref[...] = jnp.dot(g, lvW2_ref[...].astype(bf16), preferred_element_type=f32) + lvb2_ref[...]


def _mu_dec_body(x_ref, s_ref, muW1_ref, mub1_ref, muW2_ref, mub2_ref,
                 dW1_ref, drow_ref, db1_ref, dW2_ref, db2_ref,
                 rec_ref, mu_ref, mu2_ref):
    f32 = jnp.float32
    bf16 = jnp.bfloat16
    x = x_ref[...]
    h = jnp.dot(x, muW1_ref[...].astype(bf16), preferred_element_type=f32) + mub1_ref[...]
    h = _lrelu(h).astype(bf16)
    mu = jnp.dot(h, muW2_ref[...], preferred_element_type=f32) + mub2_ref[...]
    mu_ref[...] = mu
    mu2_ref[...] = mu
    t = jnp.dot(mu.astype(bf16), dW1_ref[...], preferred_element_type=f32)
    t = t + s_ref[...] * drow_ref[...] + db1_ref[...]
    t = _lrelu(t).astype(bf16)
    rec_ref[...] = jnp.dot(t, dW2_ref[...], preferred_element_type=f32) + db2_ref[...]


def kernel(data, s, mu_W1, mu_b1, mu_W2, mu_b2, lv_W1, lv_b1, lv_W2, lv_b2,
           dec_W1, dec_b1, dec_W2, dec_b2):
    n, dx = data.shape
    dh = mu_W1.shape[1]
    bn_a = 256 if n % 256 == 0 else n
    bn_b = 512 if n % 512 == 0 else n

    bf16 = jnp.bfloat16
    dW1_main = dec_W1[:dh].astype(bf16)
    drow = dec_W1[dh:dh + 1]

    row_blk = lambda i: (i, 0)
    whole = lambda i: (0, 0)
    vec = lambda i: (0,)

    lv, x16 = pl.pallas_call(
        _lv_body,
        grid=(n // bn_b,),
        in_specs=[
            pl.BlockSpec((bn_b, dx), row_blk),
            pl.BlockSpec((dx, dh), whole),
            pl.BlockSpec((dh,), vec),
            pl.BlockSpec((dh, dh), whole),
            pl.BlockSpec((dh,), vec),
        ],
        out_specs=[
            pl.BlockSpec((bn_b, dh), row_blk),
            pl.BlockSpec((bn_b, dx), row_blk),
        ],
        out_shape=[
            jax.ShapeDtypeStruct((n, dh), jnp.float32),
            jax.ShapeDtypeStruct((n, dx), bf16),
        ],
        compiler_params=pltpu.CompilerParams(
            dimension_semantics=("arbitrary",),
        ),
    )(data, lv_W1, lv_b1, lv_W2, lv_b2)

    rec, mu, mu2 = pl.pallas_call(
        _mu_dec_body,
        grid=(n // bn_a,),
        in_specs=[
            pl.BlockSpec((bn_a, dx), row_blk),
            pl.BlockSpec((bn_a, 1), row_blk),
            pl.BlockSpec((dx, dh), whole),
            pl.BlockSpec((dh,), vec),
            pl.BlockSpec((dh, dh), whole),
            pl.BlockSpec((dh,), vec),
            pl.BlockSpec((dh, dh), whole),
            pl.BlockSpec((1, dh), whole),
            pl.BlockSpec((dh,), vec),
            pl.BlockSpec((dh, dx), whole),
            pl.BlockSpec((dx,), vec),
        ],
        out_specs=[
            pl.BlockSpec((bn_a, dx), row_blk),
            pl.BlockSpec((bn_a, dh), row_blk),
            pl.BlockSpec((bn_a, dh), row_blk),
        ],
        out_shape=[
            jax.ShapeDtypeStruct((n, dx), jnp.float32),
            jax.ShapeDtypeStruct((n, dh), jnp.float32),
            jax.ShapeDtypeStruct((n, dh), jnp.float32),
        ],
        compiler_params=pltpu.CompilerParams(
            dimension_semantics=("arbitrary",),
        ),
    )(x16, s,
      mu_W1, mu_b1, mu_W2.astype(bf16), mu_b2,
      dW1_main, drow, dec_b1, dec_W2.astype(bf16), dec_b2)

    return (rec, mu, lv, mu2)


# speedup vs baseline: 1.0134x; 1.0134x over previous
import jax
import jax.numpy as jnp
from jax.experimental import pallas as pl
from jax.experimental.pallas import tpu as pltpu


def _lrelu(x):
    return jnp.maximum(x, 0.01 * x)


def _lv_body(x_ref, lvW1_ref, lvb1_ref, lvW2_ref, lvb2_ref, lv_ref, x16_ref):
    f32 = jnp.float32
    bf16 = jnp.bfloat16
    x = x_ref[...].astype(bf16)
    x16_ref[...] = x
    g = jnp.dot(x, lvW1_ref[...].astype(bf16), preferred_element_type=f32) + lvb1_ref[...]
    g = _lrelu(g).astype(bf16)
    lv_ref[...] = jnp.dot(g, lvW2_ref[...].astype(bf16), preferred_element_type=f32) + lvb2_ref[...]


def _mu_body(x_ref, muW1_ref, mub1_ref, muW2_ref, mub2_ref,
             mu_ref, mu2_ref, mu16_ref):
    f32 = jnp.float32
    bf16 = jnp.bfloat16
    x = x_ref[...]
    h = jnp.dot(x, muW1_ref[...].astype(bf16), preferred_element_type=f32) + mub1_ref[...]
    h = _lrelu(h).astype(bf16)
    mu = jnp.dot(h, muW2_ref[...].astype(bf16), preferred_element_type=f32) + mub2_ref[...]
    mu_ref[...] = mu
    mu2_ref[...] = mu
    mu16_ref[...] = mu.astype(bf16)


def _dec_body(m_ref, s_ref, dW1_ref, drow_ref, db1_ref, dW2_ref, db2_ref, rec_ref):
    f32 = jnp.float32
    bf16 = jnp.bfloat16
    m = m_ref[...]
    t = jnp.dot(m, dW1_ref[...].astype(bf16), preferred_element_type=f32)
    t = t + s_ref[...] * drow_ref[...] + db1_ref[...]
    t = _lrelu(t).astype(bf16)
    rec_ref[...] = jnp.dot(t, dW2_ref[...].astype(bf16), preferred_element_type=f32) + db2_ref[...]


def kernel(data, s, mu_W1, mu_b1, mu_W2, mu_b2, lv_W1, lv_b1, lv_W2, lv_b2,
           dec_W1, dec_b1, dec_W2, dec_b2):
    n, dx = data.shape
    dh = mu_W1.shape[1]
    bn_lv = 512 if n % 512 == 0 else n
    bn_mu = 256 if n % 256 == 0 else n
    bn_dec = 512 if n % 512 == 0 else n

    bf16 = jnp.bfloat16

    row_blk = lambda i: (i, 0)
    whole = lambda i: (0, 0)
    vec = lambda i: (0,)

    lv, x16 = pl.pallas_call(
        _lv_body,
        grid=(n // bn_lv,),
        in_specs=[
            pl.BlockSpec((bn_lv, dx), row_blk),
            pl.BlockSpec((dx, dh), whole),
            pl.BlockSpec((dh,), vec),
            pl.BlockSpec((dh, dh), whole),
            pl.BlockSpec((dh,), vec),
        ],
        out_specs=[
            pl.BlockSpec((bn_lv, dh), row_blk),
            pl.BlockSpec((bn_lv, dx), row_blk),
        ],
        out_shape=[
            jax.ShapeDtypeStruct((n, dh), jnp.float32),
            jax.ShapeDtypeStruct((n, dx), bf16),
        ],
        compiler_params=pltpu.CompilerParams(
            dimension_semantics=("arbitrary",),
        ),
    )(data, lv_W1, lv_b1, lv_W2, lv_b2)

    mu, mu2, mu16 = pl.pallas_call(
        _mu_body,
        grid=(n // bn_mu,),
        in_specs=[
            pl.BlockSpec((bn_mu, dx), row_blk),
            pl.BlockSpec((dx, dh), whole),
            pl.BlockSpec((dh,), vec),
            pl.BlockSpec((dh, dh), whole),
            pl.BlockSpec((dh,), vec),
        ],
        out_specs=[
            pl.BlockSpec((bn_mu, dh), row_blk),
            pl.BlockSpec((bn_mu, dh), row_blk),
            pl.BlockSpec((bn_mu, dh), row_blk),
        ],
        out_shape=[
            jax.ShapeDtypeStruct((n, dh), jnp.float32),
            jax.ShapeDtypeStruct((n, dh), jnp.float32),
            jax.ShapeDtypeStruct((n, dh), bf16),
        ],
        compiler_params=pltpu.CompilerParams(
            dimension_semantics=("arbitrary",),
        ),
    )(x16, mu_W1, mu_b1, mu_W2, mu_b2)

    rec = pl.pallas_call(
        _dec_body,
        grid=(n // bn_dec,),
        in_specs=[
            pl.BlockSpec((bn_dec, dh), row_blk),
            pl.BlockSpec((bn_dec, 1), row_blk),
            pl.BlockSpec((dh, dh), whole),
            pl.BlockSpec((1, dh), whole),
            pl.BlockSpec((dh,), vec),
            pl.BlockSpec((dh, dx), whole),
            pl.BlockSpec((dx,), vec),
        ],
        out_specs=pl.BlockSpec((bn_dec, dx), row_blk),
        out_shape=jax.ShapeDtypeStruct((n, dx), jnp.float32),
        compiler_params=pltpu.CompilerParams(
            dimension_semantics=("arbitrary",),
        ),
    )(mu16, s, dec_W1, dec_W1[dh:dh + 1], dec_b1, dec_W2, dec_b2)

    return (rec, mu, lv, mu2)


# speedup vs baseline: 1.0250x; 1.0115x over previous
import jax
import jax.numpy as jnp
from jax.experimental import pallas as pl
from jax.experimental.pallas import tpu as pltpu


def _lrelu(x):
    return jnp.maximum(x, 0.01 * x)


def _lv_body(x_ref, lvW1_ref, lvb1_ref, lvW2_ref, lvb2_ref, lv_ref, x16_ref):
    f32 = jnp.float32
    bf16 = jnp.bfloat16
    x = x_ref[...].astype(bf16)
    x16_ref[...] = x
    g = jnp.dot(x, lvW1_ref[...].astype(bf16), preferred_element_type=f32) + lvb1_ref[...]
    g = _lrelu(g).astype(bf16)
    lv_ref[...] = jnp.dot(g, lvW2_ref[...].astype(bf16), preferred_element_type=f32) + lvb2_ref[...]


def _mu_body(x_ref, muW1_ref, mub1_ref, muW2_ref, mub2_ref,
             mu_ref, mu2_ref, mu16_ref):
    f32 = jnp.float32
    bf16 = jnp.bfloat16
    x = x_ref[...]
    h = jnp.dot(x, muW1_ref[...].astype(bf16), preferred_element_type=f32) + mub1_ref[...]
    h = _lrelu(h).astype(bf16)
    mu = jnp.dot(h, muW2_ref[...].astype(bf16), preferred_element_type=f32) + mub2_ref[...]
    mu_ref[...] = mu
    mu2_ref[...] = mu
    mu16_ref[...] = mu.astype(bf16)


def _dec_body(m_ref, s_ref, dW1_ref, drow_ref, db1_ref, dW2_ref, db2_ref, rec_ref):
    f32 = jnp.float32
    bf16 = jnp.bfloat16
    m = m_ref[...]
    t = jnp.dot(m, dW1_ref[...].astype(bf16), preferred_element_type=f32)
    t = t + s_ref[...] * drow_ref[...] + db1_ref[...]
    t = _lrelu(t).astype(bf16)
    rec_ref[...] = jnp.dot(t, dW2_ref[...].astype(bf16), preferred_element_type=f32) + db2_ref[...]


def kernel(data, s, mu_W1, mu_b1, mu_W2, mu_b2, lv_W1, lv_b1, lv_W2, lv_b2,
           dec_W1, dec_b1, dec_W2, dec_b2):
    n, dx = data.shape
    dh = mu_W1.shape[1]
    bn_lv = 512 if n % 512 == 0 else n
    bn_mu = 512 if n % 512 == 0 else n
    bn_dec = 512 if n % 512 == 0 else n

    bf16 = jnp.bfloat16

    row_blk = lambda i: (i, 0)
    whole = lambda i: (0, 0)
    vec = lambda i: (0,)

    lv, x16 = pl.pallas_call(
        _lv_body,
        grid=(n // bn_lv,),
        in_specs=[
            pl.BlockSpec((bn_lv, dx), row_blk),
            pl.BlockSpec((dx, dh), whole),
            pl.BlockSpec((dh,), vec),
            pl.BlockSpec((dh, dh), whole),
            pl.BlockSpec((dh,), vec),
        ],
        out_specs=[
            pl.BlockSpec((bn_lv, dh), row_blk),
            pl.BlockSpec((bn_lv, dx), row_blk),
        ],
        out_shape=[
            jax.ShapeDtypeStruct((n, dh), jnp.float32),
            jax.ShapeDtypeStruct((n, dx), bf16),
        ],
        compiler_params=pltpu.CompilerParams(
            dimension_semantics=("arbitrary",),
            vmem_limit_bytes=100 * 1024 * 1024,
        ),
    )(data, lv_W1, lv_b1, lv_W2, lv_b2)

    mu, mu2, mu16 = pl.pallas_call(
        _mu_body,
        grid=(n // bn_mu,),
        in_specs=[
            pl.BlockSpec((bn_mu, dx), row_blk),
            pl.BlockSpec((dx, dh), whole),
            pl.BlockSpec((dh,), vec),
            pl.BlockSpec((dh, dh), whole),
            pl.BlockSpec((dh,), vec),
        ],
        out_specs=[
            pl.BlockSpec((bn_mu, dh), row_blk),
            pl.BlockSpec((bn_mu, dh), row_blk),
            pl.BlockSpec((bn_mu, dh), row_blk),
        ],
        out_shape=[
            jax.ShapeDtypeStruct((n, dh), jnp.float32),
            jax.ShapeDtypeStruct((n, dh), jnp.float32),
            jax.ShapeDtypeStruct((n, dh), bf16),
        ],
        compiler_params=pltpu.CompilerParams(
            dimension_semantics=("arbitrary",),
            vmem_limit_bytes=100 * 1024 * 1024,
        ),
    )(x16, mu_W1, mu_b1, mu_W2, mu_b2)

    rec = pl.pallas_call(
        _dec_body,
        grid=(n // bn_dec,),
        in_specs=[
            pl.BlockSpec((bn_dec, dh), row_blk),
            pl.BlockSpec((bn_dec, 1), row_blk),
            pl.BlockSpec((dh, dh), whole),
            pl.BlockSpec((1, dh), whole),
            pl.BlockSpec((dh,), vec),
            pl.BlockSpec((dh, dx), whole),
            pl.BlockSpec((dx,), vec),
        ],
        out_specs=pl.BlockSpec((bn_dec, dx), row_blk),
        out_shape=jax.ShapeDtypeStruct((n, dx), jnp.float32),
        compiler_params=pltpu.CompilerParams(
            dimension_semantics=("arbitrary",),
            vmem_limit_bytes=100 * 1024 * 1024,
        ),
    )(mu16, s, dec_W1, dec_W1[dh:dh + 1], dec_b1, dec_W2, dec_b2)

    return (rec, mu, lv, mu2)
